# baseline (device time: 86867 ns/iter reference)
import jax
import jax.numpy as jnp
from jax import lax
from jax.experimental import pallas as pl
from jax.experimental.pallas import tpu as pltpu

N_DEV = 8
SQ = 1024
SKV = 1024
H_PER = 8
DH = 128
D_MODEL = 1024
CHUNK = SQ // N_DEV
SCALE = 0.08838834764831843

RS_CFG = [[(4, 2), (3, 1), (1, 0)], [(3, 1), (1, 0), (4, 2)]]
AG_MASKS = [[1, 3, 4], [4, 1, 3]]


def kernel(x, Wq, K_ext, V_ext, Wo):
    my = lax.axis_index("i")
    wq_my = lax.dynamic_slice(Wq, (0, my * H_PER * DH), (D_MODEL, H_PER * DH))
    wo_my = lax.dynamic_slice(Wo, (my * H_PER * DH, 0), (H_PER * DH, D_MODEL))
    x2 = x.reshape(SQ, D_MODEL)
    k3 = K_ext.reshape(SKV, H_PER, DH)
    v3 = V_ext.reshape(SKV, H_PER, DH)

    def body(x_ref, wq_ref, k_ref, v_ref, wo_ref, out_ref,
             accA_ref, accB_ref, ctx_ref, gath_ref,
             stA0, stA1, stB0, stB1,
             rvA00, rvA10, rvA20, rvA01, rvA11, rvA21,
             rvB00, rvB10, rvB20, rvB01, rvB11, rvB21,
             sA_send, sA_recv, sB_send, sB_recv,
             ag_send_sems, ag_recv_sems):
        my_pos = lax.axis_index("i")
        b0 = my_pos & 1
        b1 = (my_pos >> 1) & 1
        b2 = (my_pos >> 2) & 1
        j1 = 4 * b1 + 2 * b0 + b2

        rvA = [[rvA00, rvA01], [rvA10, rvA11], [rvA20, rvA21]]
        rvB = [[rvB00, rvB01], [rvB10, rvB11], [rvB20, rvB21]]

        def rs_start(r, acc_ref, base, stages, recvs, ssem, rsem):
            L = 512 >> r
            rdmas = []
            newbase = []
            for p in range(2):
                m, sel = RS_CFG[p][r]
                c0 = p * 512
                b = (my_pos >> sel) & 1
                partner = my_pos ^ m
                send_off = base[p] + (1 - b) * L
                stg = stages[p]
                stg[pl.ds(0, L), :] = acc_ref[
                    pl.ds(send_off, L), c0:c0 + 512].astype(jnp.bfloat16)
                rdma = pltpu.make_async_remote_copy(
                    src_ref=stg.at[pl.ds(0, L)],
                    dst_ref=recvs[r][p],
                    send_sem=ssem.at[r, p],
                    recv_sem=rsem.at[r, p],
                    device_id=(partner,),
                    device_id_type=pl.DeviceIdType.MESH,
                )
                rdma.start()
                rdmas.append(rdma)
                newbase.append(base[p] + b * L)
            return rdmas, newbase

        def rs_finish(r, acc_ref, newbase, recvs, rdmas):
            L = 512 >> r
            for p in range(2):
                rdmas[p].wait()
            for p in range(2):
                c0 = p * 512
                acc_ref[pl.ds(newbase[p], L), c0:c0 + 512] = (
                    acc_ref[pl.ds(newbase[p], L), c0:c0 + 512]
                    + recvs[r][p][:, :].astype(jnp.float32))

        xb = x_ref[:, :].astype(jnp.bfloat16)
        wqb = wq_ref[:, :].astype(jnp.bfloat16)
        q_all = jax.lax.dot(xb, wqb, preferred_element_type=jnp.float32)
        q_all = q_all.astype(jnp.bfloat16)

        qi = lax.broadcasted_iota(jnp.int32, (SQ, SKV), 0)
        ki = lax.broadcasted_iota(jnp.int32, (SQ, SKV), 1)
        mask = (jnp.abs(qi - ki) <= 128) | (ki < 32) | (qi < 32)

        def do_head(h):
            q_h = q_all[:, h * DH:(h + 1) * DH]
            k_h = k_ref[:, h, :].astype(jnp.bfloat16)
            v_h = v_ref[:, h, :].astype(jnp.bfloat16)
            scores = lax.dot_general(
                q_h, k_h,
                dimension_numbers=(((1,), (1,)), ((), ())),
                preferred_element_type=jnp.float32,
            ) * SCALE
            scores = jnp.where(mask, scores, -1e9)
            m = jnp.max(scores, axis=1, keepdims=True)
            w = jnp.exp(scores - m)
            s = jnp.sum(w, axis=1, keepdims=True)
            w = (w / s).astype(jnp.bfloat16)
            ctx_h = jax.lax.dot(w, v_h, preferred_element_type=jnp.float32)
            ctx_ref[:, h * DH:(h + 1) * DH] = ctx_h.astype(jnp.bfloat16)

        for h in range(4):
            do_head(h)

        wob = wo_ref[:, :].astype(jnp.bfloat16)
        accA_ref[:, :] = jax.lax.dot(
            ctx_ref[:, 0:512], wob[0:512, :],
            preferred_element_type=jnp.float32)

        baseA = [my_pos * 0, my_pos * 0]
        rdA, baseA = rs_start(0, accA_ref, baseA, [stA0, stA1],
                              rvA, sA_send, sA_recv)
        do_head(4)
        rs_finish(0, accA_ref, baseA, rvA, rdA)
        rdA, baseA = rs_start(1, accA_ref, baseA, [stA0, stA1],
                              rvA, sA_send, sA_recv)
        do_head(5)
        rs_finish(1, accA_ref, baseA, rvA, rdA)
        rdA, baseA = rs_start(2, accA_ref, baseA, [stA0, stA1],
                              rvA, sA_send, sA_recv)
        do_head(6)
        do_head(7)
        rs_finish(2, accA_ref, baseA, rvA, rdA)

        accB_ref[:, :] = jax.lax.dot(
            ctx_ref[:, 512:1024], wob[512:1024, :],
            preferred_element_type=jnp.float32)
        baseB = [my_pos * 0, my_pos * 0]
        for r in range(3):
            rdB, baseB = rs_start(r, accB_ref, baseB, [stB0, stB1],
                                  rvB, sB_send, sB_recv)
            rs_finish(r, accB_ref, baseB, rvB, rdB)

        jown = [my_pos, j1]
        for p in range(2):
            c0 = p * 512
            o = jown[p] * CHUNK
            tot = (accA_ref[pl.ds(o, CHUNK), c0:c0 + 512]
                   + accB_ref[pl.ds(o, CHUNK), c0:c0 + 512])
            out_ref[0, pl.ds(o, CHUNK), c0:c0 + 512] = tot
            gath_ref[pl.ds(o, CHUNK), c0:c0 + 512] = tot.astype(jnp.bfloat16)

        for t in range(3):
            L = CHUNK << t
            rdmas = []
            pbases = []
            for p in range(2):
                m = AG_MASKS[p][t]
                c0 = p * 512
                partner = my_pos ^ m
                pb0 = partner & 1
                pb1 = (partner >> 1) & 1
                pb2 = (partner >> 2) & 1
                jp = partner if p == 0 else 4 * pb1 + 2 * pb0 + pb2
                sbase = (jown[p] & ~((1 << t) - 1)) * CHUNK
                pbase = (jp & ~((1 << t) - 1)) * CHUNK
                pbases.append(pbase)
                rdma = pltpu.make_async_remote_copy(
                    src_ref=gath_ref.at[pl.ds(sbase, L), pl.ds(c0, 512)],
                    dst_ref=gath_ref.at[pl.ds(sbase, L), pl.ds(c0, 512)],
                    send_sem=ag_send_sems.at[t, p],
                    recv_sem=ag_recv_sems.at[t, p],
                    device_id=(partner,),
                    device_id_type=pl.DeviceIdType.MESH,
                )
                rdma.start()
                rdmas.append(rdma)
            for p in range(2):
                rdmas[p].wait()
            for p in range(2):
                c0 = p * 512
                out_ref[0, pl.ds(pbases[p], L), c0:c0 + 512] = (
                    gath_ref[pl.ds(pbases[p], L), c0:c0 + 512].astype(
                        jnp.float32))

    return pl.pallas_call(
        body,
        out_shape=jax.ShapeDtypeStruct((1, SQ, D_MODEL), jnp.float32),
        in_specs=[pl.BlockSpec(memory_space=pltpu.VMEM)] * 5,
        out_specs=pl.BlockSpec(memory_space=pltpu.VMEM),
        scratch_shapes=[
            pltpu.VMEM((SQ, D_MODEL), jnp.float32),
            pltpu.VMEM((SQ, D_MODEL), jnp.float32),
            pltpu.VMEM((SQ, H_PER * DH), jnp.bfloat16),
            pltpu.VMEM((SQ, D_MODEL), jnp.bfloat16),
            pltpu.VMEM((512, 512), jnp.bfloat16),
            pltpu.VMEM((512, 512), jnp.bfloat16),
            pltpu.VMEM((512, 512), jnp.bfloat16),
            pltpu.VMEM((512, 512), jnp.bfloat16),
            pltpu.VMEM((512, 512), jnp.bfloat16),
            pltpu.VMEM((256, 512), jnp.bfloat16),
            pltpu.VMEM((128, 512), jnp.bfloat16),
            pltpu.VMEM((512, 512), jnp.bfloat16),
            pltpu.VMEM((256, 512), jnp.bfloat16),
            pltpu.VMEM((128, 512), jnp.bfloat16),
            pltpu.VMEM((512, 512), jnp.bfloat16),
            pltpu.VMEM((256, 512), jnp.bfloat16),
            pltpu.VMEM((128, 512), jnp.bfloat16),
            pltpu.VMEM((512, 512), jnp.bfloat16),
            pltpu.VMEM((256, 512), jnp.bfloat16),
            pltpu.VMEM((128, 512), jnp.bfloat16),
            pltpu.SemaphoreType.DMA((3, 2)),
            pltpu.SemaphoreType.DMA((3, 2)),
            pltpu.SemaphoreType.DMA((3, 2)),
            pltpu.SemaphoreType.DMA((3, 2)),
            pltpu.SemaphoreType.DMA((3, 2)),
            pltpu.SemaphoreType.DMA((3, 2)),
        ],
    )(x2, wq_my, k3, v3, wo_my)
